# baseline (speedup 1.0000x reference)
import jax
import jax.numpy as jnp
from jax.experimental import pallas as pl
from jax.experimental.pallas import tpu as pltpu

_NUM_CLASSES = 80
_REG_MAX = 16
_A = 5376
_G = 20
_B = 16
_K = 13
_CLS_W = 1.0
_BOX_W = 2.5


def _loss_kernel(clsT_ref, regT_ref, gb_ref, gbT_ref, ohT_ref, aprow_ref,
                 strow_ref, w_ref, out_ref):
    f32 = jnp.float32
    hi = jax.lax.Precision.HIGHEST

    gb = gb_ref[0]
    x1 = gb[:, 0:1]
    y1 = gb[:, 1:2]
    x2 = gb[:, 2:3]
    y2 = gb[:, 3:4]
    cx = (x1 + x2) * 0.5
    cy = (y1 + y2) * 0.5

    apx = aprow_ref[0:1, :]
    apy = aprow_ref[1:2, :]

    lane64 = jax.lax.broadcasted_iota(jnp.int32, (_G, 64), 1)
    jx = lane64 & 7
    jy = lane64 >> 3
    dc_parts = []
    gi_parts = []
    in_parts = []
    for s, base, n in ((32, 0, 16), (16, 256, 32), (8, 1280, 64)):
        sf = jnp.float32(s)
        i0 = jnp.clip(jnp.floor(cx * (1.0 / s)).astype(jnp.int32) - 3, 0,
                      n - 8)
        j0 = jnp.clip(jnp.floor(cy * (1.0 / s)).astype(jnp.int32) - 3, 0,
                      n - 8)
        ix = i0 + jx
        iy = j0 + jy
        candx = (ix.astype(f32) + 0.5) * sf
        candy = (iy.astype(f32) + 0.5) * sf
        ddx = candx - cx
        ddy = candy - cy
        dc_parts.append(jnp.sqrt(ddx * ddx + ddy * ddy))
        gi_parts.append(base + iy * n + ix)
        in_parts.append(((candx >= x1) & (candx <= x2) &
                         (candy >= y1) & (candy <= y2)).astype(f32))
    dc = jnp.concatenate(dc_parts, axis=1)
    gidx = jnp.concatenate(gi_parts, axis=1)
    ins_c = jnp.concatenate(in_parts, axis=1)

    big = jnp.float32(1e30)
    sels = []
    ins_t = []
    for t in range(_K):
        minv = jnp.min(dc, axis=1, keepdims=True)
        eq = dc == minv
        idx = jnp.min(jnp.where(eq, gidx, _A), axis=1, keepdims=True)
        pick = gidx == idx
        sels.append(idx)
        ins_t.append(jnp.max(jnp.where(pick, ins_c, 0.0), axis=1,
                             keepdims=True))
        dc = jnp.where(pick, big, dc)

    any_inside = ins_t[0]
    for t in range(1, _K):
        any_inside = jnp.maximum(any_inside, ins_t[t])
    any_inside = any_inside > 0.0

    lane = jax.lax.broadcasted_iota(jnp.int32, (_G, _A), 1)
    mask = jnp.zeros((_G, _A), f32)
    for t in range(_K):
        m_t = jnp.where(any_inside, ins_t[t], 1.0 if t < 3 else 0.0)
        mask = jnp.where(lane == sels[t], m_t, mask)

    giota = jax.lax.broadcasted_iota(jnp.int32, (_G, _A), 0)
    lastg = jnp.max(jnp.where(mask > 0.0, giota, -1), axis=0,
                    keepdims=True)
    oh_last = (giota == lastg).astype(f32)
    fg_row = (lastg >= 0).astype(f32)
    num_pos = jnp.sum(fg_row)

    tcT = jnp.minimum(jnp.dot(ohT_ref[0], mask), 1.0)
    tbT = jnp.dot(gbT_ref[0], oh_last, precision=hi)

    p = jnp.clip(clsT_ref[0].astype(f32), 1e-07, 1.0 - 1e-07)
    logp = jnp.log(p)
    log1mp = jnp.log(1.0 - p)
    x_log = logp - log1mp
    colneg = -jnp.dot(jnp.ones((1, _NUM_CLASSES), f32), log1mp)
    s1 = jnp.sum(tcT * x_log)
    tot_sum = jnp.sum(colneg) - s1
    pos_sum = jnp.sum(colneg * fg_row) - s1
    neg_sum = tot_sum - pos_sum

    num_neg = _A - num_pos
    safe_pos = jnp.maximum(num_pos, 1.0)
    pw = jnp.minimum(_A / safe_pos, 50.0)
    pos_loss = jnp.where(num_pos > 0, pos_sum / (safe_pos * _NUM_CLASSES) * pw,
                         0.0)
    neg_loss = neg_sum / (num_neg * _NUM_CLASSES)
    lc = pos_loss + neg_loss

    e_all = jnp.exp(regT_ref[0].astype(f32))
    sn = jnp.dot(w_ref[...], e_all)
    rds = [sn[2 * j + 1:2 * j + 2, :] / sn[2 * j:2 * j + 1, :]
           for j in range(4)]

    st = strow_ref[...]
    pb = jnp.concatenate([apx - rds[0] * st, apy - rds[1] * st,
                          apx + rds[2] * st, apy + rds[3] * st], axis=0)

    ad = jnp.abs(pb - tbT)
    sl1 = jnp.where(ad < 1.0, 0.5 * ad * ad, ad - 0.5)
    box_sum = jnp.sum(sl1 * fg_row)
    lb = jnp.where(num_pos > 0, box_sum / (safe_pos * 4.0), 0.0)

    out_ref[...] = jnp.concatenate(
        [jnp.reshape(lc, (1, 1, 1)), jnp.reshape(lb, (1, 1, 1))], axis=2)


def _sum_kernel(in_ref, out_ref):
    out_ref[...] = jnp.sum(in_ref[...], axis=0)


def _per_shard(cls_scores, reg_distri, gt_boxes, gt_labels, anchor_points,
               stride_tensor):
    nb = cls_scores.shape[0]
    onehotT = (gt_labels[:, None, :] ==
               jnp.arange(_NUM_CLASSES, dtype=gt_labels.dtype)[None, :, None]
               ).astype(jnp.float32)
    clsT = jnp.swapaxes(cls_scores.astype(jnp.bfloat16), 1, 2)
    regT = jnp.swapaxes(reg_distri.astype(jnp.bfloat16), 1, 2)
    gbT = jnp.swapaxes(gt_boxes, 1, 2)
    nbin = _REG_MAX + 1
    ch = jnp.arange(4 * nbin)
    seg = ch // nbin
    off = (ch % nbin).astype(jnp.float32)
    row = jnp.arange(8)
    w_sum = (row[:, None] == 2 * seg[None, :]).astype(jnp.float32)
    w_proj = ((row[:, None] == 2 * seg[None, :] + 1).astype(jnp.float32)
              * off[None, :])
    w_mat = w_sum + w_proj
    ap_row = anchor_points.T
    st_row = stride_tensor.T

    out = pl.pallas_call(
        _loss_kernel,
        grid=(nb,),
        in_specs=[
            pl.BlockSpec((1, _NUM_CLASSES, _A), lambda i: (i, 0, 0)),
            pl.BlockSpec((1, 4 * (_REG_MAX + 1), _A), lambda i: (i, 0, 0)),
            pl.BlockSpec((1, _G, 4), lambda i: (i, 0, 0)),
            pl.BlockSpec((1, 4, _G), lambda i: (i, 0, 0)),
            pl.BlockSpec((1, _NUM_CLASSES, _G), lambda i: (i, 0, 0)),
            pl.BlockSpec((2, _A), lambda i: (0, 0)),
            pl.BlockSpec((1, _A), lambda i: (0, 0)),
            pl.BlockSpec((8, 4 * (_REG_MAX + 1)), lambda i: (0, 0)),
        ],
        out_specs=pl.BlockSpec((1, 1, 2), lambda i: (i, 0, 0)),
        out_shape=jax.ShapeDtypeStruct((nb, 1, 2), jnp.float32),
        compiler_params=pltpu.CompilerParams(
            dimension_semantics=("parallel",)),
    )(clsT, regT, gt_boxes, gbT, onehotT, ap_row, st_row, w_mat)

    return pl.pallas_call(
        _sum_kernel,
        out_shape=jax.ShapeDtypeStruct((1, 2), jnp.float32),
    )(out)


@jax.jit
def kernel(cls_scores, reg_distri, gt_boxes, gt_labels, anchor_points,
           stride_tensor):
    out = _per_shard(cls_scores, reg_distri, gt_boxes, gt_labels,
                     anchor_points, stride_tensor)

    loss_cls = out[0, 0] / _B * _CLS_W
    loss_box = out[0, 1] / _B * _BOX_W
    loss_dfl = jnp.asarray(0.0, jnp.float32)
    total = loss_cls + loss_box + loss_dfl * 0.5
    return (total, loss_cls, loss_box, loss_dfl)


# speedup vs baseline: 1.0701x; 1.0701x over previous
import jax
import jax.numpy as jnp
from jax.experimental import pallas as pl
from jax.experimental.pallas import tpu as pltpu

_NUM_CLASSES = 80
_REG_MAX = 16
_A = 5376
_G = 20
_B = 16
_K = 13
_CLS_W = 1.0
_BOX_W = 2.5


def _loss_kernel(clsT_ref, regT_ref, gb_ref, gbT_ref, ohT_ref, aprow_ref,
                 strow_ref, w_ref, out_ref):
    f32 = jnp.float32
    hi = jax.lax.Precision.HIGHEST

    gb = gb_ref[0]
    x1 = gb[:, 0:1]
    y1 = gb[:, 1:2]
    x2 = gb[:, 2:3]
    y2 = gb[:, 3:4]
    cx = (x1 + x2) * 0.5
    cy = (y1 + y2) * 0.5

    apx = aprow_ref[0:1, :]
    apy = aprow_ref[1:2, :]

    lane64 = jax.lax.broadcasted_iota(jnp.int32, (_G, 64), 1)
    jx = lane64 & 7
    jy = lane64 >> 3
    dc_parts = []
    gi_parts = []
    in_parts = []
    for s, base, n in ((32, 0, 16), (16, 256, 32), (8, 1280, 64)):
        sf = jnp.float32(s)
        i0 = jnp.clip(jnp.floor(cx * (1.0 / s)).astype(jnp.int32) - 3, 0,
                      n - 8)
        j0 = jnp.clip(jnp.floor(cy * (1.0 / s)).astype(jnp.int32) - 3, 0,
                      n - 8)
        ix = i0 + jx
        iy = j0 + jy
        candx = (ix.astype(f32) + 0.5) * sf
        candy = (iy.astype(f32) + 0.5) * sf
        ddx = candx - cx
        ddy = candy - cy
        dc_parts.append(jnp.sqrt(ddx * ddx + ddy * ddy))
        gi_parts.append(base + iy * n + ix)
        in_parts.append(((candx >= x1) & (candx <= x2) &
                         (candy >= y1) & (candy <= y2)).astype(f32))
    dc = jnp.concatenate(dc_parts, axis=1)
    gidx = jnp.concatenate(gi_parts, axis=1)
    ins_c = jnp.concatenate(in_parts, axis=1)

    big = jnp.float32(1e30)
    sels = []
    ins_t = []
    for t in range(_K):
        minv = jnp.min(dc, axis=1, keepdims=True)
        eq = dc == minv
        idx = jnp.min(jnp.where(eq, gidx, _A), axis=1, keepdims=True)
        pick = gidx == idx
        sels.append(idx)
        ins_t.append(jnp.max(jnp.where(pick, ins_c, 0.0), axis=1,
                             keepdims=True))
        dc = jnp.where(pick, big, dc)

    any_inside = ins_t[0]
    for t in range(1, _K):
        any_inside = jnp.maximum(any_inside, ins_t[t])
    any_inside = any_inside > 0.0

    lane = jax.lax.broadcasted_iota(jnp.int32, (_G, _A), 1)
    mask = jnp.zeros((_G, _A), f32)
    for t in range(_K):
        m_t = jnp.where(any_inside, ins_t[t], 1.0 if t < 3 else 0.0)
        mask = jnp.where(lane == sels[t], m_t, mask)

    giota = jax.lax.broadcasted_iota(jnp.int32, (_G, _A), 0)
    lastg = jnp.max(jnp.where(mask > 0.0, giota, -1), axis=0,
                    keepdims=True)
    oh_last = (giota == lastg).astype(f32)
    fg_row = (lastg >= 0).astype(f32)
    num_pos = jnp.sum(fg_row)

    tcT = jnp.minimum(jnp.dot(ohT_ref[0], mask), 1.0)
    tbT = jnp.dot(gbT_ref[0], oh_last, precision=hi)

    p = jnp.clip(clsT_ref[0], 1e-07, 1.0 - 1e-07)
    logp = jnp.log(p)
    log1mp = jnp.log(1.0 - p)
    x_log = logp - log1mp
    colneg = -jnp.dot(jnp.ones((1, _NUM_CLASSES), f32), log1mp)
    s1 = jnp.sum(tcT * x_log)
    tot_sum = jnp.sum(colneg) - s1
    pos_sum = jnp.sum(colneg * fg_row) - s1
    neg_sum = tot_sum - pos_sum

    num_neg = _A - num_pos
    safe_pos = jnp.maximum(num_pos, 1.0)
    pw = jnp.minimum(_A / safe_pos, 50.0)
    pos_loss = jnp.where(num_pos > 0, pos_sum / (safe_pos * _NUM_CLASSES) * pw,
                         0.0)
    neg_loss = neg_sum / (num_neg * _NUM_CLASSES)
    lc = pos_loss + neg_loss

    e_all = jnp.exp(regT_ref[0])
    sn = jnp.dot(w_ref[...], e_all)
    rds = [sn[2 * j + 1:2 * j + 2, :] / sn[2 * j:2 * j + 1, :]
           for j in range(4)]

    st = strow_ref[...]
    pb = jnp.concatenate([apx - rds[0] * st, apy - rds[1] * st,
                          apx + rds[2] * st, apy + rds[3] * st], axis=0)

    ad = jnp.abs(pb - tbT)
    sl1 = jnp.where(ad < 1.0, 0.5 * ad * ad, ad - 0.5)
    box_sum = jnp.sum(sl1 * fg_row)
    lb = jnp.where(num_pos > 0, box_sum / (safe_pos * 4.0), 0.0)

    out_ref[...] = jnp.concatenate(
        [jnp.reshape(lc, (1, 1, 1)), jnp.reshape(lb, (1, 1, 1))], axis=2)


def _sum_kernel(in_ref, out_ref):
    out_ref[...] = jnp.sum(in_ref[...], axis=0)


def _per_shard(cls_scores, reg_distri, gt_boxes, gt_labels, anchor_points,
               stride_tensor):
    nb = cls_scores.shape[0]
    onehotT = (gt_labels[:, None, :] ==
               jnp.arange(_NUM_CLASSES, dtype=gt_labels.dtype)[None, :, None]
               ).astype(jnp.float32)
    clsT = jnp.swapaxes(cls_scores, 1, 2)
    regT = jnp.swapaxes(reg_distri, 1, 2)
    gbT = jnp.swapaxes(gt_boxes, 1, 2)
    nbin = _REG_MAX + 1
    ch = jnp.arange(4 * nbin)
    seg = ch // nbin
    off = (ch % nbin).astype(jnp.float32)
    row = jnp.arange(8)
    w_sum = (row[:, None] == 2 * seg[None, :]).astype(jnp.float32)
    w_proj = ((row[:, None] == 2 * seg[None, :] + 1).astype(jnp.float32)
              * off[None, :])
    w_mat = w_sum + w_proj
    ap_row = anchor_points.T
    st_row = stride_tensor.T

    out = pl.pallas_call(
        _loss_kernel,
        grid=(nb,),
        in_specs=[
            pl.BlockSpec((1, _NUM_CLASSES, _A), lambda i: (i, 0, 0)),
            pl.BlockSpec((1, 4 * (_REG_MAX + 1), _A), lambda i: (i, 0, 0)),
            pl.BlockSpec((1, _G, 4), lambda i: (i, 0, 0)),
            pl.BlockSpec((1, 4, _G), lambda i: (i, 0, 0)),
            pl.BlockSpec((1, _NUM_CLASSES, _G), lambda i: (i, 0, 0)),
            pl.BlockSpec((2, _A), lambda i: (0, 0)),
            pl.BlockSpec((1, _A), lambda i: (0, 0)),
            pl.BlockSpec((8, 4 * (_REG_MAX + 1)), lambda i: (0, 0)),
        ],
        out_specs=pl.BlockSpec((1, 1, 2), lambda i: (i, 0, 0)),
        out_shape=jax.ShapeDtypeStruct((nb, 1, 2), jnp.float32),
        compiler_params=pltpu.CompilerParams(
            dimension_semantics=("parallel",)),
    )(clsT, regT, gt_boxes, gbT, onehotT, ap_row, st_row, w_mat)

    return pl.pallas_call(
        _sum_kernel,
        out_shape=jax.ShapeDtypeStruct((1, 2), jnp.float32),
    )(out)


@jax.jit
def kernel(cls_scores, reg_distri, gt_boxes, gt_labels, anchor_points,
           stride_tensor):
    out = _per_shard(cls_scores, reg_distri, gt_boxes, gt_labels,
                     anchor_points, stride_tensor)

    loss_cls = out[0, 0] / _B * _CLS_W
    loss_box = out[0, 1] / _B * _BOX_W
    loss_dfl = jnp.asarray(0.0, jnp.float32)
    total = loss_cls + loss_box + loss_dfl * 0.5
    return (total, loss_cls, loss_box, loss_dfl)
